# Initial kernel scaffold; baseline (speedup 1.0000x reference)
import jax
import jax.numpy as jnp
from jax.experimental import pallas as pl


def kernel(x, edge_index, edge_attr, batch_ids, mol_idx, y, W_self, W_nbr, W_edge, b):
    raise NotImplementedError("write your pallas kernel here")


# speedup vs baseline: 3.9467x; 3.9467x over previous
import functools

import jax
import jax.numpy as jnp
from jax import lax
from jax.experimental import pallas as pl
from jax.experimental.pallas import tpu as pltpu
from jax.experimental.pallas import tpu_sc as plsc

N_NODES = 10000
N_EDGES = 320000
D_FEAT = 128
D_EDGE = 16
BATCH = 128

NC = 2
NS = 16
NW = NC * NS
EPW = N_EDGES // NW
K = 80
NCHUNK = EPW // K
RPT = 640
RPT_LAST = N_NODES - 15 * RPT

def _sc_edge_agg_body(sd_hbm, x_hbm, ea_hbm, z1_hbm, z2_hbm,
                     s1_out, s2_out,
                     sd_v, rows_v, ea_v, s1_sh, s2_sh, sem):
    c = lax.axis_index("c")
    s = lax.axis_index("s")
    wid = c * NS + s

    @pl.when(s < NS - 1)
    def _():
        pltpu.sync_copy(z1_hbm, s1_sh.at[pl.ds(s * RPT, RPT)])
        pltpu.sync_copy(z2_hbm, s2_sh.at[pl.ds(s * RPT, RPT)])

    @pl.when(s == NS - 1)
    def _():
        pltpu.sync_copy(z1_hbm.at[pl.ds(0, RPT_LAST)],
                        s1_sh.at[pl.ds(s * RPT, RPT_LAST)])
        pltpu.sync_copy(z2_hbm.at[pl.ds(0, RPT_LAST)],
                        s2_sh.at[pl.ds(s * RPT, RPT_LAST)])

    plsc.subcore_barrier()

    def chunk(j, carry):
        pltpu.sync_copy(sd_hbm.at[wid, j], sd_v)
        pltpu.async_copy(x_hbm.at[sd_v.at[0]], rows_v, sem).wait()
        pltpu.sync_copy(rows_v, s1_sh.at[sd_v.at[1]], add=True)
        pltpu.sync_copy(ea_hbm.at[wid, j], ea_v)
        pltpu.sync_copy(ea_v, s2_sh.at[sd_v.at[1]], add=True)
        return carry

    lax.fori_loop(0, NCHUNK, chunk, 0)
    plsc.subcore_barrier()

    @pl.when(s < NS - 1)
    def _():
        pltpu.sync_copy(s1_sh.at[pl.ds(s * RPT, RPT)],
                        s1_out.at[c, pl.ds(s * RPT, RPT)])
        pltpu.sync_copy(s2_sh.at[pl.ds(s * RPT, RPT)],
                        s2_out.at[c, pl.ds(s * RPT, RPT)])

    @pl.when(s == NS - 1)
    def _():
        pltpu.sync_copy(s1_sh.at[pl.ds(s * RPT, RPT_LAST)],
                        s1_out.at[c, pl.ds(s * RPT, RPT_LAST)])
        pltpu.sync_copy(s2_sh.at[pl.ds(s * RPT, RPT_LAST)],
                        s2_out.at[c, pl.ds(s * RPT, RPT_LAST)])


@functools.cache
def _sc_edge_agg():
    mesh = plsc.VectorSubcoreMesh(core_axis_name="c", subcore_axis_name="s",
                                  num_cores=NC, num_subcores=NS)
    return pl.kernel(
        _sc_edge_agg_body,
        out_type=[
            jax.ShapeDtypeStruct((NC, N_NODES, D_FEAT), jnp.float32),
            jax.ShapeDtypeStruct((NC, N_NODES, D_EDGE), jnp.float32),
        ],
        mesh=mesh,
        compiler_params=pltpu.CompilerParams(use_tc_tiling_on_sc=False),
        scratch_types=[
            pltpu.VMEM((2, K), jnp.int32),
            pltpu.VMEM((K, D_FEAT), jnp.float32),
            pltpu.VMEM((K, D_EDGE), jnp.float32),
            pltpu.VMEM_SHARED((N_NODES, D_FEAT), jnp.float32),
            pltpu.VMEM_SHARED((N_NODES, D_EDGE), jnp.float32),
            pltpu.SemaphoreType.DMA,
        ],
    )


BLK = 1000
NBLK = N_NODES // BLK


def _tc_body(x_ref, s1_ref, s2_ref, bid_ref, mol_ref,
             wself_ref, wnbr_ref, wedge_ref, b_ref, out_ref, acc_ref):
    i = pl.program_id(0)

    @pl.when(i == 0)
    def _():
        acc_ref[...] = jnp.zeros_like(acc_ref)

    s1 = s1_ref[0] + s1_ref[1]
    s2 = s2_ref[0] + s2_ref[1]
    nr = (jnp.dot(x_ref[...], wself_ref[...], preferred_element_type=jnp.float32)
          + jnp.dot(s1, wnbr_ref[...], preferred_element_type=jnp.float32)
          + jnp.dot(s2, wedge_ref[...], preferred_element_type=jnp.float32)
          + b_ref[...])
    nr = jnp.maximum(nr, 0.0)

    seg = bid_ref[0, 0, :] * 2 + mol_ref[0, 0, :]
    oh = (seg[:, None] == lax.broadcasted_iota(jnp.int32, (BLK, 2 * BATCH), 1))
    acc_ref[...] += lax.dot_general(oh.astype(jnp.float32), nr,
                                    (((0,), (0,)), ((), ())),
                                    preferred_element_type=jnp.float32)

    @pl.when(i == NBLK - 1)
    def _():
        out_ref[...] = acc_ref[...]


_tc_pool = pl.pallas_call(
    _tc_body,
    grid=(NBLK,),
    in_specs=[
        pl.BlockSpec((BLK, D_FEAT), lambda i: (i, 0)),
        pl.BlockSpec((NC, BLK, D_FEAT), lambda i: (0, i, 0)),
        pl.BlockSpec((NC, BLK, D_EDGE), lambda i: (0, i, 0)),
        pl.BlockSpec((1, 1, BLK), lambda i: (i, 0, 0)),
        pl.BlockSpec((1, 1, BLK), lambda i: (i, 0, 0)),
        pl.BlockSpec((D_FEAT, D_FEAT), lambda i: (0, 0)),
        pl.BlockSpec((D_FEAT, D_FEAT), lambda i: (0, 0)),
        pl.BlockSpec((D_EDGE, D_FEAT), lambda i: (0, 0)),
        pl.BlockSpec((1, D_FEAT), lambda i: (0, 0)),
    ],
    out_specs=pl.BlockSpec((2 * BATCH, D_FEAT), lambda i: (0, 0)),
    out_shape=jax.ShapeDtypeStruct((2 * BATCH, D_FEAT), jnp.float32),
    scratch_shapes=[pltpu.VMEM((2 * BATCH, D_FEAT), jnp.float32)],
)


@jax.jit
def kernel(x, edge_index, edge_attr, batch_ids, mol_idx, y,
           W_self, W_nbr, W_edge, b):
    sd = jnp.stack([edge_index[0].astype(jnp.int32).reshape(NW, NCHUNK, K),
                    edge_index[1].astype(jnp.int32).reshape(NW, NCHUNK, K)],
                   axis=2)
    ea = edge_attr.reshape(NW, NCHUNK, K, D_EDGE)
    z1 = jnp.zeros((RPT, D_FEAT), jnp.float32)
    z2 = jnp.zeros((RPT, D_EDGE), jnp.float32)

    s1p, s2p = _sc_edge_agg()(sd, x, ea, z1, z2)

    bid = batch_ids.astype(jnp.int32).reshape(NBLK, 1, BLK)
    mol = mol_idx.astype(jnp.int32).reshape(NBLK, 1, BLK)
    pooled = _tc_pool(x, s1p, s2p, bid, mol,
                      W_self, W_nbr, W_edge, b.reshape(1, D_FEAT))
    return pooled.reshape(BATCH, 2 * D_FEAT), y


# speedup vs baseline: 6.1615x; 1.5612x over previous
import functools

import jax
import jax.numpy as jnp
from jax import lax
from jax.experimental import pallas as pl
from jax.experimental.pallas import tpu as pltpu
from jax.experimental.pallas import tpu_sc as plsc

N_NODES = 10000
N_EDGES = 320000
D_FEAT = 128
D_EDGE = 16
BATCH = 128

NC = 2
NS = 16
NW = NC * NS
EPW = N_EDGES // NW
K = 80
NCHUNK = EPW // K
RPT = 640
RPT_LAST = N_NODES - 15 * RPT

def _sc_edge_agg_body(sd_hbm, x_hbm, ea_hbm, z1_hbm, z2_hbm,
                      s1_out, s2_out,
                      sd0, sd1, r0, r1, e0, e1, s1_sh, s2_sh,
                      g0, g1, a0, a1, p0, p1, q0, q1):
    c = lax.axis_index("c")
    s = lax.axis_index("s")
    wid = c * NS + s

    SD, RW, EA = (sd0, sd1), (r0, r1), (e0, e1)
    GS, AS, PS, QS = (g0, g1), (a0, a1), (p0, p1), (q0, q1)

    @pl.when(s < NS - 1)
    def _():
        pltpu.sync_copy(z1_hbm, s1_sh.at[pl.ds(s * RPT, RPT)])
        pltpu.sync_copy(z2_hbm, s2_sh.at[pl.ds(s * RPT, RPT)])

    @pl.when(s == NS - 1)
    def _():
        pltpu.sync_copy(z1_hbm.at[pl.ds(0, RPT_LAST)],
                        s1_sh.at[pl.ds(s * RPT, RPT_LAST)])
        pltpu.sync_copy(z2_hbm.at[pl.ds(0, RPT_LAST)],
                        s2_sh.at[pl.ds(s * RPT, RPT_LAST)])

    plsc.subcore_barrier()

    def idx_load(j, b):
        pltpu.sync_copy(sd_hbm.at[wid, j], SD[b])

    def fe_start(j, b):
        pltpu.async_copy(x_hbm.at[SD[b].at[0]], RW[b], GS[b])
        pltpu.async_copy(ea_hbm.at[wid, j], EA[b], AS[b])

    def gather_wait(b):
        pltpu.make_async_copy(x_hbm.at[SD[b].at[0]], RW[b], GS[b]).wait()

    def ea_wait(j, b):
        pltpu.make_async_copy(ea_hbm.at[wid, j], EA[b], AS[b]).wait()

    def scat_start(b):
        pltpu.async_copy(RW[b], s1_sh.at[SD[b].at[1]], PS[b], add=True)
        pltpu.async_copy(EA[b], s2_sh.at[SD[b].at[1]], QS[b], add=True)

    def scat_wait(b):
        pltpu.make_async_copy(RW[b], s1_sh.at[SD[b].at[1]], PS[b]).wait()
        pltpu.make_async_copy(EA[b], s2_sh.at[SD[b].at[1]], QS[b]).wait()

    for b in (0, 1):
        idx_load(b, b)
        fe_start(b, b)

    def body(g, carry):
        j0 = 2 * g
        for b in (0, 1):
            gather_wait(b)
            ea_wait(j0 + b, b)
            scat_start(b)
        for b in (0, 1):
            scat_wait(b)
            jn = j0 + b + 2

            @pl.when(jn < NCHUNK)
            def _():
                idx_load(jn, b)
                fe_start(jn, b)
        return carry

    lax.fori_loop(0, NCHUNK // 2, body, 0)

    if NCHUNK % 2:
        gather_wait(0)
        ea_wait(NCHUNK - 1, 0)
        scat_start(0)
        scat_wait(0)

    plsc.subcore_barrier()

    @pl.when(s < NS - 1)
    def _():
        pltpu.sync_copy(s1_sh.at[pl.ds(s * RPT, RPT)],
                        s1_out.at[c, pl.ds(s * RPT, RPT)])
        pltpu.sync_copy(s2_sh.at[pl.ds(s * RPT, RPT)],
                        s2_out.at[c, pl.ds(s * RPT, RPT)])

    @pl.when(s == NS - 1)
    def _():
        pltpu.sync_copy(s1_sh.at[pl.ds(s * RPT, RPT_LAST)],
                        s1_out.at[c, pl.ds(s * RPT, RPT_LAST)])
        pltpu.sync_copy(s2_sh.at[pl.ds(s * RPT, RPT_LAST)],
                        s2_out.at[c, pl.ds(s * RPT, RPT_LAST)])


@functools.cache
def _sc_edge_agg():
    mesh = plsc.VectorSubcoreMesh(core_axis_name="c", subcore_axis_name="s",
                                  num_cores=NC, num_subcores=NS)
    return pl.kernel(
        _sc_edge_agg_body,
        out_type=[
            jax.ShapeDtypeStruct((NC, N_NODES, D_FEAT), jnp.float32),
            jax.ShapeDtypeStruct((NC, N_NODES, D_EDGE), jnp.float32),
        ],
        mesh=mesh,
        compiler_params=pltpu.CompilerParams(use_tc_tiling_on_sc=False),
        scratch_types=(
            [pltpu.VMEM((2, K), jnp.int32)] * 2
            + [pltpu.VMEM((K, D_FEAT), jnp.float32)] * 2
            + [pltpu.VMEM((K, D_EDGE), jnp.float32)] * 2
            + [pltpu.VMEM_SHARED((N_NODES, D_FEAT), jnp.float32),
               pltpu.VMEM_SHARED((N_NODES, D_EDGE), jnp.float32)]
            + [pltpu.SemaphoreType.DMA] * 8
        ),
    )


BLK = 1000
NBLK = N_NODES // BLK


def _tc_body(x_ref, s1_ref, s2_ref, bid_ref, mol_ref,
             wself_ref, wnbr_ref, wedge_ref, b_ref, out_ref, acc_ref):
    i = pl.program_id(0)

    @pl.when(i == 0)
    def _():
        acc_ref[...] = jnp.zeros_like(acc_ref)

    s1 = s1_ref[0] + s1_ref[1]
    s2 = s2_ref[0] + s2_ref[1]
    nr = (jnp.dot(x_ref[...], wself_ref[...], preferred_element_type=jnp.float32)
          + jnp.dot(s1, wnbr_ref[...], preferred_element_type=jnp.float32)
          + jnp.dot(s2, wedge_ref[...], preferred_element_type=jnp.float32)
          + b_ref[...])
    nr = jnp.maximum(nr, 0.0)

    seg = bid_ref[0, 0, :] * 2 + mol_ref[0, 0, :]
    oh = (seg[:, None] == lax.broadcasted_iota(jnp.int32, (BLK, 2 * BATCH), 1))
    acc_ref[...] += lax.dot_general(oh.astype(jnp.float32), nr,
                                    (((0,), (0,)), ((), ())),
                                    preferred_element_type=jnp.float32)

    @pl.when(i == NBLK - 1)
    def _():
        out_ref[...] = acc_ref[...]


_tc_pool = pl.pallas_call(
    _tc_body,
    grid=(NBLK,),
    in_specs=[
        pl.BlockSpec((BLK, D_FEAT), lambda i: (i, 0)),
        pl.BlockSpec((NC, BLK, D_FEAT), lambda i: (0, i, 0)),
        pl.BlockSpec((NC, BLK, D_EDGE), lambda i: (0, i, 0)),
        pl.BlockSpec((1, 1, BLK), lambda i: (i, 0, 0)),
        pl.BlockSpec((1, 1, BLK), lambda i: (i, 0, 0)),
        pl.BlockSpec((D_FEAT, D_FEAT), lambda i: (0, 0)),
        pl.BlockSpec((D_FEAT, D_FEAT), lambda i: (0, 0)),
        pl.BlockSpec((D_EDGE, D_FEAT), lambda i: (0, 0)),
        pl.BlockSpec((1, D_FEAT), lambda i: (0, 0)),
    ],
    out_specs=pl.BlockSpec((2 * BATCH, D_FEAT), lambda i: (0, 0)),
    out_shape=jax.ShapeDtypeStruct((2 * BATCH, D_FEAT), jnp.float32),
    scratch_shapes=[pltpu.VMEM((2 * BATCH, D_FEAT), jnp.float32)],
)


@jax.jit
def kernel(x, edge_index, edge_attr, batch_ids, mol_idx, y,
           W_self, W_nbr, W_edge, b):
    sd = jnp.stack([edge_index[0].astype(jnp.int32).reshape(NW, NCHUNK, K),
                    edge_index[1].astype(jnp.int32).reshape(NW, NCHUNK, K)],
                   axis=2)
    ea = edge_attr.reshape(NW, NCHUNK, K, D_EDGE)
    z1 = jnp.zeros((RPT, D_FEAT), jnp.float32)
    z2 = jnp.zeros((RPT, D_EDGE), jnp.float32)

    s1p, s2p = _sc_edge_agg()(sd, x, ea, z1, z2)

    bid = batch_ids.astype(jnp.int32).reshape(NBLK, 1, BLK)
    mol = mol_idx.astype(jnp.int32).reshape(NBLK, 1, BLK)
    pooled = _tc_pool(x, s1p, s2p, bid, mol,
                      W_self, W_nbr, W_edge, b.reshape(1, D_FEAT))
    return pooled.reshape(BATCH, 2 * D_FEAT), y


# speedup vs baseline: 6.5050x; 1.0557x over previous
import functools

import jax
import jax.numpy as jnp
from jax import lax
from jax.experimental import pallas as pl
from jax.experimental.pallas import tpu as pltpu
from jax.experimental.pallas import tpu_sc as plsc

N_NODES = 10000
N_EDGES = 320000
D_FEAT = 128
D_EDGE = 16
BATCH = 128

NC = 2
NS = 16
NW = NC * NS
EPW = N_EDGES // NW
K = 80
NCHUNK = EPW // K
RPT = 640
RPT_LAST = N_NODES - 15 * RPT

def _sc_edge_agg_body(sd_hbm, x_hbm, ea_hbm, z1_hbm, z2_hbm,
                      s1_out, s2_out,
                      sd0, sd1, r0, r1, e0, e1, s1_sh, s2_sh,
                      g0, g1, a0, a1, p0, p1, q0, q1):
    c = lax.axis_index("c")
    s = lax.axis_index("s")
    wid = c * NS + s

    SD, RW, EA = (sd0, sd1), (r0, r1), (e0, e1)
    GS, AS, PS, QS = (g0, g1), (a0, a1), (p0, p1), (q0, q1)

    @pl.when(s < NS - 1)
    def _():
        pltpu.sync_copy(z1_hbm, s1_sh.at[pl.ds(s * RPT, RPT)])
        pltpu.sync_copy(z2_hbm, s2_sh.at[pl.ds(s * RPT, RPT)])

    @pl.when(s == NS - 1)
    def _():
        pltpu.sync_copy(z1_hbm.at[pl.ds(0, RPT_LAST)],
                        s1_sh.at[pl.ds(s * RPT, RPT_LAST)])
        pltpu.sync_copy(z2_hbm.at[pl.ds(0, RPT_LAST)],
                        s2_sh.at[pl.ds(s * RPT, RPT_LAST)])

    plsc.subcore_barrier()

    ebase = wid * EPW

    def idx_load(j, b):
        pltpu.sync_copy(sd_hbm.at[:, pl.ds(ebase + j * K, K)], SD[b])

    def fe_start(j, b):
        pltpu.async_copy(x_hbm.at[SD[b].at[0]], RW[b], GS[b])
        pltpu.async_copy(ea_hbm.at[pl.ds(ebase + j * K, K)], EA[b], AS[b])

    def gather_wait(b):
        pltpu.make_async_copy(x_hbm.at[SD[b].at[0]], RW[b], GS[b]).wait()

    def ea_wait(j, b):
        pltpu.make_async_copy(ea_hbm.at[pl.ds(ebase + j * K, K)],
                              EA[b], AS[b]).wait()

    def scat_start(b):
        pltpu.async_copy(RW[b], s1_sh.at[SD[b].at[1]], PS[b], add=True)
        pltpu.async_copy(EA[b], s2_sh.at[SD[b].at[1]], QS[b], add=True)

    def scat_wait(b):
        pltpu.make_async_copy(RW[b], s1_sh.at[SD[b].at[1]], PS[b]).wait()
        pltpu.make_async_copy(EA[b], s2_sh.at[SD[b].at[1]], QS[b]).wait()

    for b in (0, 1):
        idx_load(b, b)
        fe_start(b, b)

    def body(g, carry):
        j0 = 2 * g
        for b in (0, 1):
            gather_wait(b)
            ea_wait(j0 + b, b)
            scat_start(b)
        for b in (0, 1):
            scat_wait(b)
            jn = j0 + b + 2

            @pl.when(jn < NCHUNK)
            def _():
                idx_load(jn, b)
                fe_start(jn, b)
        return carry

    lax.fori_loop(0, NCHUNK // 2, body, 0)

    if NCHUNK % 2:
        gather_wait(0)
        ea_wait(NCHUNK - 1, 0)
        scat_start(0)
        scat_wait(0)

    plsc.subcore_barrier()

    @pl.when(s < NS - 1)
    def _():
        pltpu.sync_copy(s1_sh.at[pl.ds(s * RPT, RPT)],
                        s1_out.at[c, pl.ds(s * RPT, RPT)])
        pltpu.sync_copy(s2_sh.at[pl.ds(s * RPT, RPT)],
                        s2_out.at[c, pl.ds(s * RPT, RPT)])

    @pl.when(s == NS - 1)
    def _():
        pltpu.sync_copy(s1_sh.at[pl.ds(s * RPT, RPT_LAST)],
                        s1_out.at[c, pl.ds(s * RPT, RPT_LAST)])
        pltpu.sync_copy(s2_sh.at[pl.ds(s * RPT, RPT_LAST)],
                        s2_out.at[c, pl.ds(s * RPT, RPT_LAST)])


@functools.cache
def _sc_edge_agg():
    mesh = plsc.VectorSubcoreMesh(core_axis_name="c", subcore_axis_name="s",
                                  num_cores=NC, num_subcores=NS)
    return pl.kernel(
        _sc_edge_agg_body,
        out_type=[
            jax.ShapeDtypeStruct((NC, N_NODES, D_FEAT), jnp.float32),
            jax.ShapeDtypeStruct((NC, N_NODES, D_EDGE), jnp.float32),
        ],
        mesh=mesh,
        compiler_params=pltpu.CompilerParams(use_tc_tiling_on_sc=False),
        scratch_types=(
            [pltpu.VMEM((2, K), jnp.int32)] * 2
            + [pltpu.VMEM((K, D_FEAT), jnp.float32)] * 2
            + [pltpu.VMEM((K, D_EDGE), jnp.float32)] * 2
            + [pltpu.VMEM_SHARED((N_NODES, D_FEAT), jnp.float32),
               pltpu.VMEM_SHARED((N_NODES, D_EDGE), jnp.float32)]
            + [pltpu.SemaphoreType.DMA] * 8
        ),
    )


BLK = 1000
NBLK = N_NODES // BLK


def _tc_body(x_ref, s1_ref, s2_ref, bid_ref, mol_ref,
             wself_ref, wnbr_ref, wedge_ref, b_ref, out_ref, acc_ref):
    i = pl.program_id(0)

    @pl.when(i == 0)
    def _():
        acc_ref[...] = jnp.zeros_like(acc_ref)

    s1 = s1_ref[0] + s1_ref[1]
    s2 = s2_ref[0] + s2_ref[1]
    nr = (jnp.dot(x_ref[...], wself_ref[...], preferred_element_type=jnp.float32)
          + jnp.dot(s1, wnbr_ref[...], preferred_element_type=jnp.float32)
          + jnp.dot(s2, wedge_ref[...], preferred_element_type=jnp.float32)
          + b_ref[...])
    nr = jnp.maximum(nr, 0.0)

    seg = bid_ref[0, 0, :] * 2 + mol_ref[0, 0, :]
    oh = (seg[:, None] == lax.broadcasted_iota(jnp.int32, (BLK, 2 * BATCH), 1))
    acc_ref[...] += lax.dot_general(oh.astype(jnp.float32), nr,
                                    (((0,), (0,)), ((), ())),
                                    preferred_element_type=jnp.float32)

    @pl.when(i == NBLK - 1)
    def _():
        out_ref[...] = acc_ref[...]


_tc_pool = pl.pallas_call(
    _tc_body,
    grid=(NBLK,),
    in_specs=[
        pl.BlockSpec((BLK, D_FEAT), lambda i: (i, 0)),
        pl.BlockSpec((NC, BLK, D_FEAT), lambda i: (0, i, 0)),
        pl.BlockSpec((NC, BLK, D_EDGE), lambda i: (0, i, 0)),
        pl.BlockSpec((1, 1, BLK), lambda i: (i, 0, 0)),
        pl.BlockSpec((1, 1, BLK), lambda i: (i, 0, 0)),
        pl.BlockSpec((D_FEAT, D_FEAT), lambda i: (0, 0)),
        pl.BlockSpec((D_FEAT, D_FEAT), lambda i: (0, 0)),
        pl.BlockSpec((D_EDGE, D_FEAT), lambda i: (0, 0)),
        pl.BlockSpec((1, D_FEAT), lambda i: (0, 0)),
    ],
    out_specs=pl.BlockSpec((2 * BATCH, D_FEAT), lambda i: (0, 0)),
    out_shape=jax.ShapeDtypeStruct((2 * BATCH, D_FEAT), jnp.float32),
    scratch_shapes=[pltpu.VMEM((2 * BATCH, D_FEAT), jnp.float32)],
)


@jax.jit
def kernel(x, edge_index, edge_attr, batch_ids, mol_idx, y,
           W_self, W_nbr, W_edge, b):
    sd = edge_index.astype(jnp.int32)
    z1 = jnp.zeros((RPT, D_FEAT), jnp.float32)
    z2 = jnp.zeros((RPT, D_EDGE), jnp.float32)

    s1p, s2p = _sc_edge_agg()(sd, x, edge_attr, z1, z2)

    bid = batch_ids.astype(jnp.int32).reshape(NBLK, 1, BLK)
    mol = mol_idx.astype(jnp.int32).reshape(NBLK, 1, BLK)
    pooled = _tc_pool(x, s1p, s2p, bid, mol,
                      W_self, W_nbr, W_edge, b.reshape(1, D_FEAT))
    return pooled.reshape(BATCH, 2 * D_FEAT), y


# speedup vs baseline: 8.1774x; 1.2571x over previous
import functools

import jax
import jax.numpy as jnp
from jax import lax
from jax.experimental import pallas as pl
from jax.experimental.pallas import tpu as pltpu
from jax.experimental.pallas import tpu_sc as plsc

N_NODES = 10000
N_EDGES = 320000
D_FEAT = 128
D_EDGE = 16
BATCH = 128

NC = 2
NS = 16
NW = NC * NS
EPW = N_EDGES // NW
K = 80
NCHUNK = EPW // K
RPT = 640
RPT_LAST = N_NODES - 15 * RPT

def _sc_edge_agg_body(sd_hbm, x_hbm, ea_hbm, z1_hbm, z2_hbm,
                      s1_out, s2_out,
                      sd0, sd1, r0, r1, et0, et1, e0, e1, s1_sh, s2_sh,
                      g0, g1, a0, a1, p0, p1, q0, q1):
    c = lax.axis_index("c")
    s = lax.axis_index("s")
    wid = c * NS + s

    SD, RW, EA, EAT = (sd0, sd1), (r0, r1), (e0, e1), (et0, et1)
    GS, AS, PS, QS = (g0, g1), (a0, a1), (p0, p1), (q0, q1)

    @pl.when(s < NS - 1)
    def _():
        pltpu.sync_copy(z1_hbm, s1_sh.at[pl.ds(s * RPT, RPT)])
        pltpu.sync_copy(z2_hbm, s2_sh.at[pl.ds(s * RPT, RPT)])

    @pl.when(s == NS - 1)
    def _():
        pltpu.sync_copy(z1_hbm.at[pl.ds(0, RPT_LAST)],
                        s1_sh.at[pl.ds(s * RPT, RPT_LAST)])
        pltpu.sync_copy(z2_hbm.at[pl.ds(0, RPT_LAST)],
                        s2_sh.at[pl.ds(s * RPT, RPT_LAST)])

    plsc.subcore_barrier()

    ebase = wid * EPW

    def idx_load(j, b):
        pltpu.sync_copy(sd_hbm.at[:, pl.ds(ebase + j * K, K)], SD[b])

    def fe_start(j, b):
        pltpu.async_copy(x_hbm.at[SD[b].at[0]], RW[b], GS[b])
        pltpu.async_copy(ea_hbm.at[:, pl.ds(ebase + j * K, K)], EAT[b], AS[b])

    def gather_wait(b):
        pltpu.make_async_copy(x_hbm.at[SD[b].at[0]], RW[b], GS[b]).wait()

    def ea_wait(j, b):
        pltpu.make_async_copy(ea_hbm.at[:, pl.ds(ebase + j * K, K)],
                              EAT[b], AS[b]).wait()

    feat_ids = lax.iota(jnp.int32, 16)

    def ea_transpose(b):
        def blk(e8, carry):
            for u in range(8):
                e = e8 * 8 + u
                eidx = lax.broadcast(e, (16,)).astype(jnp.int32)
                EA[b][e, :] = plsc.load_gather(EAT[b], [feat_ids, eidx])
            return carry
        lax.fori_loop(0, K // 8, blk, 0)

    def scat_start(b):
        pltpu.async_copy(RW[b], s1_sh.at[SD[b].at[1]], PS[b], add=True)
        pltpu.async_copy(EA[b], s2_sh.at[SD[b].at[1]], QS[b], add=True)

    def scat_wait(b):
        pltpu.make_async_copy(RW[b], s1_sh.at[SD[b].at[1]], PS[b]).wait()
        pltpu.make_async_copy(EA[b], s2_sh.at[SD[b].at[1]], QS[b]).wait()

    for b in (0, 1):
        idx_load(b, b)
        fe_start(b, b)

    def body(g, carry):
        j0 = 2 * g
        for b in (0, 1):
            gather_wait(b)
            ea_wait(j0 + b, b)
            ea_transpose(b)
            scat_start(b)
        for b in (0, 1):
            scat_wait(b)
            jn = j0 + b + 2

            @pl.when(jn < NCHUNK)
            def _():
                idx_load(jn, b)
                fe_start(jn, b)
        return carry

    lax.fori_loop(0, NCHUNK // 2, body, 0)

    if NCHUNK % 2:
        gather_wait(0)
        ea_wait(NCHUNK - 1, 0)
        ea_transpose(0)
        scat_start(0)
        scat_wait(0)

    plsc.subcore_barrier()

    @pl.when(s < NS - 1)
    def _():
        pltpu.sync_copy(s1_sh.at[pl.ds(s * RPT, RPT)],
                        s1_out.at[c, pl.ds(s * RPT, RPT)])
        pltpu.sync_copy(s2_sh.at[pl.ds(s * RPT, RPT)],
                        s2_out.at[c, pl.ds(s * RPT, RPT)])

    @pl.when(s == NS - 1)
    def _():
        pltpu.sync_copy(s1_sh.at[pl.ds(s * RPT, RPT_LAST)],
                        s1_out.at[c, pl.ds(s * RPT, RPT_LAST)])
        pltpu.sync_copy(s2_sh.at[pl.ds(s * RPT, RPT_LAST)],
                        s2_out.at[c, pl.ds(s * RPT, RPT_LAST)])


@functools.cache
def _sc_edge_agg():
    mesh = plsc.VectorSubcoreMesh(core_axis_name="c", subcore_axis_name="s",
                                  num_cores=NC, num_subcores=NS)
    return pl.kernel(
        _sc_edge_agg_body,
        out_type=[
            jax.ShapeDtypeStruct((NC, N_NODES, D_FEAT), jnp.float32),
            jax.ShapeDtypeStruct((NC, N_NODES, D_EDGE), jnp.float32),
        ],
        mesh=mesh,
        compiler_params=pltpu.CompilerParams(use_tc_tiling_on_sc=False,
                                             needs_layout_passes=False),
        scratch_types=(
            [pltpu.VMEM((2, K), jnp.int32)] * 2
            + [pltpu.VMEM((K, D_FEAT), jnp.float32)] * 2
            + [pltpu.VMEM((D_EDGE, K), jnp.float32)] * 2
            + [pltpu.VMEM((K, D_EDGE), jnp.float32)] * 2
            + [pltpu.VMEM_SHARED((N_NODES, D_FEAT), jnp.float32),
               pltpu.VMEM_SHARED((N_NODES, D_EDGE), jnp.float32)]
            + [pltpu.SemaphoreType.DMA] * 8
        ),
    )


BLK = 1000
NBLK = N_NODES // BLK


def _tc_body(x_ref, s1_ref, s2_ref, bid_ref, mol_ref,
             wself_ref, wnbr_ref, wedge_ref, b_ref, out_ref, acc_ref):
    i = pl.program_id(0)

    @pl.when(i == 0)
    def _():
        acc_ref[...] = jnp.zeros_like(acc_ref)

    s1 = s1_ref[0] + s1_ref[1]
    s2 = s2_ref[0] + s2_ref[1]
    nr = (jnp.dot(x_ref[...], wself_ref[...], preferred_element_type=jnp.float32)
          + jnp.dot(s1, wnbr_ref[...], preferred_element_type=jnp.float32)
          + jnp.dot(s2, wedge_ref[...], preferred_element_type=jnp.float32)
          + b_ref[...])
    nr = jnp.maximum(nr, 0.0)

    seg = bid_ref[0, 0, :] * 2 + mol_ref[0, 0, :]
    oh = (seg[:, None] == lax.broadcasted_iota(jnp.int32, (BLK, 2 * BATCH), 1))
    acc_ref[...] += lax.dot_general(oh.astype(jnp.float32), nr,
                                    (((0,), (0,)), ((), ())),
                                    preferred_element_type=jnp.float32)

    @pl.when(i == NBLK - 1)
    def _():
        out_ref[...] = acc_ref[...]


_tc_pool = pl.pallas_call(
    _tc_body,
    grid=(NBLK,),
    in_specs=[
        pl.BlockSpec((BLK, D_FEAT), lambda i: (i, 0)),
        pl.BlockSpec((NC, BLK, D_FEAT), lambda i: (0, i, 0)),
        pl.BlockSpec((NC, BLK, D_EDGE), lambda i: (0, i, 0)),
        pl.BlockSpec((1, 1, BLK), lambda i: (i, 0, 0)),
        pl.BlockSpec((1, 1, BLK), lambda i: (i, 0, 0)),
        pl.BlockSpec((D_FEAT, D_FEAT), lambda i: (0, 0)),
        pl.BlockSpec((D_FEAT, D_FEAT), lambda i: (0, 0)),
        pl.BlockSpec((D_EDGE, D_FEAT), lambda i: (0, 0)),
        pl.BlockSpec((1, D_FEAT), lambda i: (0, 0)),
    ],
    out_specs=pl.BlockSpec((2 * BATCH, D_FEAT), lambda i: (0, 0)),
    out_shape=jax.ShapeDtypeStruct((2 * BATCH, D_FEAT), jnp.float32),
    scratch_shapes=[pltpu.VMEM((2 * BATCH, D_FEAT), jnp.float32)],
)


@jax.jit
def kernel(x, edge_index, edge_attr, batch_ids, mol_idx, y,
           W_self, W_nbr, W_edge, b):
    sd = edge_index.astype(jnp.int32)
    z1 = jnp.zeros((RPT, D_FEAT), jnp.float32)
    z2 = jnp.zeros((RPT, D_EDGE), jnp.float32)

    s1p, s2p = _sc_edge_agg()(sd, x, edge_attr.T, z1, z2)

    bid = batch_ids.astype(jnp.int32).reshape(NBLK, 1, BLK)
    mol = mol_idx.astype(jnp.int32).reshape(NBLK, 1, BLK)
    pooled = _tc_pool(x, s1p, s2p, bid, mol,
                      W_self, W_nbr, W_edge, b.reshape(1, D_FEAT))
    return pooled.reshape(BATCH, 2 * D_FEAT), y


# speedup vs baseline: 8.1981x; 1.0025x over previous
import functools

import jax
import jax.numpy as jnp
from jax import lax
from jax.experimental import pallas as pl
from jax.experimental.pallas import tpu as pltpu
from jax.experimental.pallas import tpu_sc as plsc

N_NODES = 10000
N_EDGES = 320000
D_FEAT = 128
D_EDGE = 16
BATCH = 128

NC = 2
NS = 16
NW = NC * NS
EPW = N_EDGES // NW
K = 80
NCHUNK = EPW // K
RPT = 640
RPT_LAST = N_NODES - 15 * RPT

def _sc_edge_agg_body(sd_hbm, x_hbm, ea_hbm, z1_hbm, z2_hbm,
                      s1_out, s2_out,
                      sd0, sd1, r0, r1, et0, et1, e0, e1, s1_sh, s2_sh,
                      g0, g1, a0, a1, p0, p1, q0, q1):
    c = lax.axis_index("c")
    s = lax.axis_index("s")
    wid = c * NS + s

    SD, RW, EA, EAT = (sd0, sd1), (r0, r1), (e0, e1), (et0, et1)
    GS, AS, PS, QS = (g0, g1), (a0, a1), (p0, p1), (q0, q1)

    @pl.when(s < NS - 1)
    def _():
        pltpu.sync_copy(z1_hbm, s1_sh.at[pl.ds(s * RPT, RPT)])
        pltpu.sync_copy(z2_hbm, s2_sh.at[pl.ds(s * RPT, RPT)])

    @pl.when(s == NS - 1)
    def _():
        pltpu.sync_copy(z1_hbm.at[pl.ds(0, RPT_LAST)],
                        s1_sh.at[pl.ds(s * RPT, RPT_LAST)])
        pltpu.sync_copy(z2_hbm.at[pl.ds(0, RPT_LAST)],
                        s2_sh.at[pl.ds(s * RPT, RPT_LAST)])

    plsc.subcore_barrier()

    ebase = wid * EPW

    def idx_load(j, b):
        pltpu.sync_copy(sd_hbm.at[:, pl.ds(ebase + j * K, K)], SD[b])

    def fe_start(j, b):
        pltpu.async_copy(x_hbm.at[SD[b].at[0]], RW[b], GS[b])
        pltpu.async_copy(ea_hbm.at[:, pl.ds(ebase + j * K, K)], EAT[b], AS[b])

    def gather_wait(b):
        pltpu.make_async_copy(x_hbm.at[SD[b].at[0]], RW[b], GS[b]).wait()

    def ea_wait(j, b):
        pltpu.make_async_copy(ea_hbm.at[:, pl.ds(ebase + j * K, K)],
                              EAT[b], AS[b]).wait()

    feat_ids = lax.iota(jnp.int32, 16)

    def ea_transpose(b):
        def blk(e8, carry):
            for u in range(8):
                e = e8 * 8 + u
                eidx = lax.broadcast(e, (16,)).astype(jnp.int32)
                EA[b][e, :] = plsc.load_gather(EAT[b], [feat_ids, eidx])
            return carry
        lax.fori_loop(0, K // 8, blk, 0)
        for e in range(K - K % 8, K):
            eidx = lax.broadcast(e, (16,)).astype(jnp.int32)
            EA[b][e, :] = plsc.load_gather(EAT[b], [feat_ids, eidx])

    def scat1_start(b):
        pltpu.async_copy(RW[b], s1_sh.at[SD[b].at[1]], PS[b], add=True)

    def scat2_start(b):
        pltpu.async_copy(EA[b], s2_sh.at[SD[b].at[1]], QS[b], add=True)

    def scat_wait(b):
        pltpu.make_async_copy(RW[b], s1_sh.at[SD[b].at[1]], PS[b]).wait()
        pltpu.make_async_copy(EA[b], s2_sh.at[SD[b].at[1]], QS[b]).wait()

    for b in (0, 1):
        idx_load(b, b)
        fe_start(b, b)

    def body(g, carry):
        j0 = 2 * g
        for b in (0, 1):
            gather_wait(b)
            scat1_start(b)
            ea_wait(j0 + b, b)
            ea_transpose(b)
            scat2_start(b)
        for b in (0, 1):
            scat_wait(b)
            jn = j0 + b + 2

            @pl.when(jn < NCHUNK)
            def _():
                idx_load(jn, b)
                fe_start(jn, b)
        return carry

    lax.fori_loop(0, NCHUNK // 2, body, 0)

    if NCHUNK % 2:
        gather_wait(0)
        scat1_start(0)
        ea_wait(NCHUNK - 1, 0)
        ea_transpose(0)
        scat2_start(0)
        scat_wait(0)

    plsc.subcore_barrier()

    @pl.when(s < NS - 1)
    def _():
        pltpu.sync_copy(s1_sh.at[pl.ds(s * RPT, RPT)],
                        s1_out.at[c, pl.ds(s * RPT, RPT)])
        pltpu.sync_copy(s2_sh.at[pl.ds(s * RPT, RPT)],
                        s2_out.at[c, pl.ds(s * RPT, RPT)])

    @pl.when(s == NS - 1)
    def _():
        pltpu.sync_copy(s1_sh.at[pl.ds(s * RPT, RPT_LAST)],
                        s1_out.at[c, pl.ds(s * RPT, RPT_LAST)])
        pltpu.sync_copy(s2_sh.at[pl.ds(s * RPT, RPT_LAST)],
                        s2_out.at[c, pl.ds(s * RPT, RPT_LAST)])


@functools.cache
def _sc_edge_agg():
    mesh = plsc.VectorSubcoreMesh(core_axis_name="c", subcore_axis_name="s",
                                  num_cores=NC, num_subcores=NS)
    return pl.kernel(
        _sc_edge_agg_body,
        out_type=[
            jax.ShapeDtypeStruct((NC, N_NODES, D_FEAT), jnp.float32),
            jax.ShapeDtypeStruct((NC, N_NODES, D_EDGE), jnp.float32),
        ],
        mesh=mesh,
        compiler_params=pltpu.CompilerParams(use_tc_tiling_on_sc=False,
                                             needs_layout_passes=False),
        scratch_types=(
            [pltpu.VMEM((2, K), jnp.int32)] * 2
            + [pltpu.VMEM((K, D_FEAT), jnp.float32)] * 2
            + [pltpu.VMEM((D_EDGE, K), jnp.float32)] * 2
            + [pltpu.VMEM((K, D_EDGE), jnp.float32)] * 2
            + [pltpu.VMEM_SHARED((N_NODES, D_FEAT), jnp.float32),
               pltpu.VMEM_SHARED((N_NODES, D_EDGE), jnp.float32)]
            + [pltpu.SemaphoreType.DMA] * 8
        ),
    )


BLK = 1000
NBLK = N_NODES // BLK


def _tc_body(x_ref, s1_ref, s2_ref, bid_ref, mol_ref,
             wself_ref, wnbr_ref, wedge_ref, b_ref, out_ref, acc_ref):
    i = pl.program_id(0)

    @pl.when(i == 0)
    def _():
        acc_ref[...] = jnp.zeros_like(acc_ref)

    s1 = s1_ref[0] + s1_ref[1]
    s2 = s2_ref[0] + s2_ref[1]
    nr = (jnp.dot(x_ref[...], wself_ref[...], preferred_element_type=jnp.float32)
          + jnp.dot(s1, wnbr_ref[...], preferred_element_type=jnp.float32)
          + jnp.dot(s2, wedge_ref[...], preferred_element_type=jnp.float32)
          + b_ref[...])
    nr = jnp.maximum(nr, 0.0)

    seg = bid_ref[0, 0, :] * 2 + mol_ref[0, 0, :]
    oh = (seg[:, None] == lax.broadcasted_iota(jnp.int32, (BLK, 2 * BATCH), 1))
    acc_ref[...] += lax.dot_general(oh.astype(jnp.float32), nr,
                                    (((0,), (0,)), ((), ())),
                                    preferred_element_type=jnp.float32)

    @pl.when(i == NBLK - 1)
    def _():
        out_ref[...] = acc_ref[...]


_tc_pool = pl.pallas_call(
    _tc_body,
    grid=(NBLK,),
    in_specs=[
        pl.BlockSpec((BLK, D_FEAT), lambda i: (i, 0)),
        pl.BlockSpec((NC, BLK, D_FEAT), lambda i: (0, i, 0)),
        pl.BlockSpec((NC, BLK, D_EDGE), lambda i: (0, i, 0)),
        pl.BlockSpec((1, 1, BLK), lambda i: (i, 0, 0)),
        pl.BlockSpec((1, 1, BLK), lambda i: (i, 0, 0)),
        pl.BlockSpec((D_FEAT, D_FEAT), lambda i: (0, 0)),
        pl.BlockSpec((D_FEAT, D_FEAT), lambda i: (0, 0)),
        pl.BlockSpec((D_EDGE, D_FEAT), lambda i: (0, 0)),
        pl.BlockSpec((1, D_FEAT), lambda i: (0, 0)),
    ],
    out_specs=pl.BlockSpec((2 * BATCH, D_FEAT), lambda i: (0, 0)),
    out_shape=jax.ShapeDtypeStruct((2 * BATCH, D_FEAT), jnp.float32),
    scratch_shapes=[pltpu.VMEM((2 * BATCH, D_FEAT), jnp.float32)],
)


@jax.jit
def kernel(x, edge_index, edge_attr, batch_ids, mol_idx, y,
           W_self, W_nbr, W_edge, b):
    sd = edge_index.astype(jnp.int32)
    z1 = jnp.zeros((RPT, D_FEAT), jnp.float32)
    z2 = jnp.zeros((RPT, D_EDGE), jnp.float32)

    s1p, s2p = _sc_edge_agg()(sd, x, edge_attr.T, z1, z2)

    bid = batch_ids.astype(jnp.int32).reshape(NBLK, 1, BLK)
    mol = mol_idx.astype(jnp.int32).reshape(NBLK, 1, BLK)
    pooled = _tc_pool(x, s1p, s2p, bid, mol,
                      W_self, W_nbr, W_edge, b.reshape(1, D_FEAT))
    return pooled.reshape(BATCH, 2 * D_FEAT), y


# speedup vs baseline: 8.2839x; 1.0105x over previous
import functools

import jax
import jax.numpy as jnp
from jax import lax
from jax.experimental import pallas as pl
from jax.experimental.pallas import tpu as pltpu
from jax.experimental.pallas import tpu_sc as plsc

N_NODES = 10000
N_EDGES = 320000
D_FEAT = 128
D_EDGE = 16
BATCH = 128

NC = 2
NS = 16
NW = NC * NS
EPW = N_EDGES // NW
K = 80
NCHUNK = EPW // K
RPT = 640
RPT_LAST = N_NODES - 15 * RPT

def _sc_edge_agg_body(sd_hbm, x_hbm, ea_hbm, z1_hbm, z2_hbm,
                      s1_out, s2_out,
                      sd0, sd1, r0, r1, et0, et1, e0, e1, s1_sh, s2_sh,
                      g0, g1, a0, a1, p0, p1, q0, q1):
    c = lax.axis_index("c")
    s = lax.axis_index("s")
    wid = c * NS + s

    SD, RW, EA, EAT = (sd0, sd1), (r0, r1), (e0, e1), (et0, et1)
    GS, AS, PS, QS = (g0, g1), (a0, a1), (p0, p1), (q0, q1)

    @pl.when(s < NS - 1)
    def _():
        pltpu.sync_copy(z1_hbm, s1_sh.at[pl.ds(s * RPT, RPT)])
        pltpu.sync_copy(z2_hbm, s2_sh.at[pl.ds(s * RPT, RPT)])

    @pl.when(s == NS - 1)
    def _():
        pltpu.sync_copy(z1_hbm.at[pl.ds(0, RPT_LAST)],
                        s1_sh.at[pl.ds(s * RPT, RPT_LAST)])
        pltpu.sync_copy(z2_hbm.at[pl.ds(0, RPT_LAST)],
                        s2_sh.at[pl.ds(s * RPT, RPT_LAST)])

    plsc.subcore_barrier()

    ebase = wid * EPW

    def idx_load(j, b):
        pltpu.sync_copy(sd_hbm.at[:, pl.ds(ebase + j * K, K)], SD[b])

    def fe_start(j, b):
        pltpu.async_copy(x_hbm.at[SD[b].at[0]], RW[b], GS[b])
        pltpu.async_copy(ea_hbm.at[:, pl.ds(ebase + j * K, K)], EAT[b], AS[b])

    def gather_wait(b):
        pltpu.make_async_copy(x_hbm.at[SD[b].at[0]], RW[b], GS[b]).wait()

    def ea_wait(j, b):
        pltpu.make_async_copy(ea_hbm.at[:, pl.ds(ebase + j * K, K)],
                              EAT[b], AS[b]).wait()

    feat_ids = lax.iota(jnp.int32, 16)

    def ea_transpose(b):
        def blk(e8, carry):
            for u in range(8):
                e = e8 * 8 + u
                eidx = lax.broadcast(e, (16,)).astype(jnp.int32)
                EA[b][e, :] = plsc.load_gather(EAT[b], [feat_ids, eidx])
            return carry
        lax.fori_loop(0, K // 8, blk, 0)
        for e in range(K - K % 8, K):
            eidx = lax.broadcast(e, (16,)).astype(jnp.int32)
            EA[b][e, :] = plsc.load_gather(EAT[b], [feat_ids, eidx])

    def scat1_start(b):
        pltpu.async_copy(RW[b], s1_sh.at[SD[b].at[1]], PS[b], add=True)

    def scat2_start(b):
        pltpu.async_copy(EA[b], s2_sh.at[SD[b].at[1]], QS[b], add=True)

    def scat_wait(b):
        pltpu.make_async_copy(RW[b], s1_sh.at[SD[b].at[1]], PS[b]).wait()
        pltpu.make_async_copy(EA[b], s2_sh.at[SD[b].at[1]], QS[b]).wait()

    for b in (0, 1):
        idx_load(b, b)
        fe_start(b, b)

    def body(g, carry):
        j0 = 2 * g
        for b in (0, 1):
            gather_wait(b)
            scat1_start(b)
            ea_wait(j0 + b, b)
            ea_transpose(b)
            scat2_start(b)
        for b in (0, 1):
            scat_wait(b)
            jn = j0 + b + 2

            @pl.when(jn < NCHUNK)
            def _():
                idx_load(jn, b)
                fe_start(jn, b)
        return carry

    lax.fori_loop(0, NCHUNK // 2, body, 0)

    if NCHUNK % 2:
        gather_wait(0)
        scat1_start(0)
        ea_wait(NCHUNK - 1, 0)
        ea_transpose(0)
        scat2_start(0)
        scat_wait(0)

    plsc.subcore_barrier()

    @pl.when(s < NS - 1)
    def _():
        pltpu.sync_copy(s1_sh.at[pl.ds(s * RPT, RPT)],
                        s1_out.at[c, pl.ds(s * RPT, RPT)])
        pltpu.sync_copy(s2_sh.at[pl.ds(s * RPT, RPT)],
                        s2_out.at[c, pl.ds(s * RPT, RPT)])

    @pl.when(s == NS - 1)
    def _():
        pltpu.sync_copy(s1_sh.at[pl.ds(s * RPT, RPT_LAST)],
                        s1_out.at[c, pl.ds(s * RPT, RPT_LAST)])
        pltpu.sync_copy(s2_sh.at[pl.ds(s * RPT, RPT_LAST)],
                        s2_out.at[c, pl.ds(s * RPT, RPT_LAST)])


@functools.cache
def _sc_edge_agg():
    mesh = plsc.VectorSubcoreMesh(core_axis_name="c", subcore_axis_name="s",
                                  num_cores=NC, num_subcores=NS)
    return pl.kernel(
        _sc_edge_agg_body,
        out_type=[
            jax.ShapeDtypeStruct((NC, N_NODES, D_FEAT), jnp.float32),
            jax.ShapeDtypeStruct((NC, N_NODES, D_EDGE), jnp.float32),
        ],
        mesh=mesh,
        compiler_params=pltpu.CompilerParams(use_tc_tiling_on_sc=False,
                                             needs_layout_passes=False),
        scratch_types=(
            [pltpu.VMEM((2, K), jnp.int32)] * 2
            + [pltpu.VMEM((K, D_FEAT), jnp.float32)] * 2
            + [pltpu.VMEM((D_EDGE, K), jnp.float32)] * 2
            + [pltpu.VMEM((K, D_EDGE), jnp.float32)] * 2
            + [pltpu.VMEM_SHARED((N_NODES, D_FEAT), jnp.float32),
               pltpu.VMEM_SHARED((N_NODES, D_EDGE), jnp.float32)]
            + [pltpu.SemaphoreType.DMA] * 8
        ),
    )


BLK = 2000
NBLK = N_NODES // BLK


def _tc_body(x_ref, s1_ref, s2_ref, bid_ref, mol_ref,
             wself_ref, wnbr_ref, wedge_ref, b_ref, out_ref, acc_ref):
    i = pl.program_id(0)

    @pl.when(i == 0)
    def _():
        acc_ref[...] = jnp.zeros_like(acc_ref)

    s1 = s1_ref[0] + s1_ref[1]
    s2 = s2_ref[0] + s2_ref[1]
    nr = (jnp.dot(x_ref[...], wself_ref[...], preferred_element_type=jnp.float32)
          + jnp.dot(s1, wnbr_ref[...], preferred_element_type=jnp.float32)
          + jnp.dot(s2, wedge_ref[...], preferred_element_type=jnp.float32)
          + b_ref[...])
    nr = jnp.maximum(nr, 0.0)

    seg = bid_ref[0, 0, :] * 2 + mol_ref[0, 0, :]
    oh = (seg[:, None] == lax.broadcasted_iota(jnp.int32, (BLK, 2 * BATCH), 1))
    acc_ref[...] += lax.dot_general(oh.astype(jnp.float32), nr,
                                    (((0,), (0,)), ((), ())),
                                    preferred_element_type=jnp.float32)

    @pl.when(i == NBLK - 1)
    def _():
        out_ref[...] = acc_ref[...]


_tc_pool = pl.pallas_call(
    _tc_body,
    grid=(NBLK,),
    in_specs=[
        pl.BlockSpec((BLK, D_FEAT), lambda i: (i, 0)),
        pl.BlockSpec((NC, BLK, D_FEAT), lambda i: (0, i, 0)),
        pl.BlockSpec((NC, BLK, D_EDGE), lambda i: (0, i, 0)),
        pl.BlockSpec((1, 1, BLK), lambda i: (i, 0, 0)),
        pl.BlockSpec((1, 1, BLK), lambda i: (i, 0, 0)),
        pl.BlockSpec((D_FEAT, D_FEAT), lambda i: (0, 0)),
        pl.BlockSpec((D_FEAT, D_FEAT), lambda i: (0, 0)),
        pl.BlockSpec((D_EDGE, D_FEAT), lambda i: (0, 0)),
        pl.BlockSpec((1, D_FEAT), lambda i: (0, 0)),
    ],
    out_specs=pl.BlockSpec((2 * BATCH, D_FEAT), lambda i: (0, 0)),
    out_shape=jax.ShapeDtypeStruct((2 * BATCH, D_FEAT), jnp.float32),
    scratch_shapes=[pltpu.VMEM((2 * BATCH, D_FEAT), jnp.float32)],
)


@jax.jit
def kernel(x, edge_index, edge_attr, batch_ids, mol_idx, y,
           W_self, W_nbr, W_edge, b):
    sd = edge_index.astype(jnp.int32)
    z1 = jnp.zeros((RPT, D_FEAT), jnp.float32)
    z2 = jnp.zeros((RPT, D_EDGE), jnp.float32)

    s1p, s2p = _sc_edge_agg()(sd, x, edge_attr.T, z1, z2)

    bid = batch_ids.astype(jnp.int32).reshape(NBLK, 1, BLK)
    mol = mol_idx.astype(jnp.int32).reshape(NBLK, 1, BLK)
    pooled = _tc_pool(x, s1p, s2p, bid, mol,
                      W_self, W_nbr, W_edge, b.reshape(1, D_FEAT))
    return pooled.reshape(BATCH, 2 * D_FEAT), y
